# baseline (speedup 1.0000x reference)
import functools

import jax
import jax.numpy as jnp
from jax import lax
from jax.experimental import pallas as pl
from jax.experimental.pallas import tpu as pltpu
from jax.experimental.pallas import tpu_sc as plsc

NUM_ROWS = 100000
EMB_K = 16
BATCH = 16384

NC = 2
NS = 16
NW = NC * NS
BPW = BATCH // NW
IDX_BLK = 128
NIDX = BPW // IDX_BLK

TCH = 800
NCHUNK = NUM_ROWS // TCH
CPW = -(-NCHUNK // NW)

PACK = 128 // EMB_K
PB = BATCH // PACK
MLP_BLK = 512


def _sc_transpose_body(wt_hbm, ht_hbm, a_hbm, b_hbm, colbuf, rowbuf, sem):
    wid = lax.axis_index("s") * NC + lax.axis_index("c")
    lanes = lax.iota(jnp.int32, EMB_K) * TCH

    def do_table(src_hbm, dst_hbm, c):
        base = c * TCH
        for j in range(EMB_K):
            pltpu.sync_copy(src_hbm.at[j, pl.ds(base, TCH)],
                            colbuf.at[pl.ds(j * TCH, TCH)])

        def row_step(r, carry):
            row = plsc.load_gather(colbuf, [lanes + r])
            rowbuf[r] = row
            return carry

        lax.fori_loop(0, TCH, row_step, 0)
        pltpu.sync_copy(rowbuf, dst_hbm.at[pl.ds(base, TCH)])

    def chunk_step(i, carry):
        c = i * NW + wid

        @pl.when(c < NCHUNK)
        def _():
            do_table(wt_hbm, a_hbm, c)
            do_table(ht_hbm, b_hbm, c)

        return carry

    lax.fori_loop(0, CPW, chunk_step, 0)


def _sc_gather_body(a_hbm, b_hbm, uidx_hbm, vidx_hbm, u_hbm, v_hbm,
                    uidx_v, vidx_v, arows_v, brows_v, sem_a, sem_b):
    wid = lax.axis_index("s") * NC + lax.axis_index("c")
    base = wid * BPW

    pltpu.sync_copy(uidx_hbm.at[wid], uidx_v)
    pltpu.sync_copy(vidx_hbm.at[wid], vidx_v)

    copies = []
    for j in range(NIDX):
        dst = pl.ds(j * IDX_BLK, IDX_BLK)
        copies.append(pltpu.async_copy(a_hbm.at[uidx_v.at[j]], arows_v.at[dst], sem_a))
        copies.append(pltpu.async_copy(b_hbm.at[vidx_v.at[j]], brows_v.at[dst], sem_b))
    for c in copies:
        c.wait()

    pltpu.sync_copy(arows_v, u_hbm.at[pl.ds(base, BPW)])
    pltpu.sync_copy(brows_v, v_hbm.at[pl.ds(base, BPW)])


@functools.cache
def _get_sc_transpose():
    dst_sds = jax.ShapeDtypeStruct((NUM_ROWS, EMB_K), jnp.float32)
    return pl.kernel(
        _sc_transpose_body,
        out_type=(dst_sds, dst_sds),
        mesh=plsc.VectorSubcoreMesh(core_axis_name="c", subcore_axis_name="s",
                                    num_cores=NC, num_subcores=NS),
        compiler_params=pltpu.CompilerParams(needs_layout_passes=False,
                                             use_tc_tiling_on_sc=False),
        scratch_types=[
            pltpu.VMEM((EMB_K * TCH,), jnp.float32),
            pltpu.VMEM((TCH, EMB_K), jnp.float32),
            pltpu.SemaphoreType.DMA,
        ],
    )


@functools.cache
def _get_sc_gather():
    emb_sds = jax.ShapeDtypeStruct((BATCH, EMB_K), jnp.float32)
    return pl.kernel(
        _sc_gather_body,
        out_type=(emb_sds, emb_sds),
        mesh=plsc.VectorSubcoreMesh(core_axis_name="c", subcore_axis_name="s",
                                    num_cores=NC, num_subcores=NS),
        compiler_params=pltpu.CompilerParams(needs_layout_passes=False,
                                             use_tc_tiling_on_sc=False),
        scratch_types=[
            pltpu.VMEM((NIDX, IDX_BLK), jnp.int32),
            pltpu.VMEM((NIDX, IDX_BLK), jnp.int32),
            pltpu.VMEM((BPW, EMB_K), jnp.float32),
            pltpu.VMEM((BPW, EMB_K), jnp.float32),
            pltpu.SemaphoreType.DMA,
            pltpu.SemaphoreType.DMA,
        ],
    )


def _mlp_body(u_ref, v_ref, bdu_ref, bdv_ref, b1t_ref, bd2_ref, o_ref):
    h = (
        jnp.dot(u_ref[...], bdu_ref[...], preferred_element_type=jnp.float32)
        + jnp.dot(v_ref[...], bdv_ref[...], preferred_element_type=jnp.float32)
        + b1t_ref[...]
    )
    h = jnp.maximum(h, 0.0)
    o_ref[...] = jnp.dot(h, bd2_ref[...], preferred_element_type=jnp.float32)


def _mlp(u2, v2, bdu, bdv, b1t, bd2):
    blk = pl.BlockSpec((MLP_BLK, 128), lambda i: (i, 0))
    w128 = pl.BlockSpec((128, 128), lambda i: (0, 0))
    row = pl.BlockSpec((1, 128), lambda i: (0, 0))
    w8 = pl.BlockSpec((128, PACK), lambda i: (0, 0))
    oblk = pl.BlockSpec((MLP_BLK, PACK), lambda i: (i, 0))
    return pl.pallas_call(
        _mlp_body,
        grid=(PB // MLP_BLK,),
        in_specs=[blk, blk, w128, w128, row, w8],
        out_specs=oblk,
        out_shape=jax.ShapeDtypeStruct((PB, PACK), jnp.float32),
    )(u2, v2, bdu, bdv, b1t, bd2)


@jax.jit
def _ncf_forward(x, W, H, W1, b1, W2):
    A, B = _get_sc_transpose()(W.T, H.T)

    uidx = x[:, 0].astype(jnp.int32).reshape(NW, NIDX, IDX_BLK)
    vidx = x[:, 1].astype(jnp.int32).reshape(NW, NIDX, IDX_BLK)
    U, V = _get_sc_gather()(A, B, uidx, vidx)

    w1ut = W1[:, :EMB_K].T
    w1vt = W1[:, EMB_K:].T
    eye = jnp.eye(PACK, dtype=jnp.float32)
    bdu = jnp.einsum("tb,jk->tjbk", eye, w1ut).reshape(128, 128)
    bdv = jnp.einsum("tb,jk->tjbk", eye, w1vt).reshape(128, 128)
    b1t = jnp.tile(b1.reshape(1, EMB_K), (1, PACK))
    bd2 = jnp.einsum("tb,k->tkb", eye, W2.reshape(EMB_K)).reshape(128, PACK)

    out = _mlp(U.reshape(PB, 128), V.reshape(PB, 128), bdu, bdv, b1t, bd2)
    return out.reshape(BATCH, 1)


def kernel(x, W, H, W1, b1, W2):
    return _ncf_forward(x, W, H, W1, b1, W2)


# speedup vs baseline: 1.5479x; 1.5479x over previous
import functools

import jax
import jax.numpy as jnp
from jax import lax
from jax.experimental import pallas as pl
from jax.experimental.pallas import tpu as pltpu
from jax.experimental.pallas import tpu_sc as plsc

NUM_ROWS = 100000
EMB_K = 16
BATCH = 16384

NC = 2
NS = 16
NW = NC * NS
BPW = BATCH // NW
IDX_BLK = 128
NIDX = BPW // IDX_BLK

TCH = 800
NCHUNK = NUM_ROWS // TCH
CPW = -(-NCHUNK // NW)

PACK = 128 // EMB_K
PB = BATCH // PACK
MLP_BLK = 512


def _sc_transpose_body(wt_hbm, ht_hbm, a_hbm, b_hbm,
                       colbuf0, colbuf1, rowbuf0, rowbuf1,
                       sem_in, sem_out0, sem_out1):
    wid = lax.axis_index("s") * NC + lax.axis_index("c")
    jlanes = lax.iota(jnp.int32, EMB_K)
    colbufs = (colbuf0, colbuf1)
    rowbufs = (rowbuf0, rowbuf1)
    out_sems = (sem_out0, sem_out1)

    steps = []
    for t, (src, dst) in enumerate(((wt_hbm, a_hbm), (ht_hbm, b_hbm))):
        for i in range(CPW):
            steps.append((t * CPW + i, src, dst, i))

    def chunk_base(i):
        c = i * NW + wid
        c = jnp.where(c < NCHUNK, c, 0)
        return c * TCH

    in_copies = {}
    out_copies = {}
    for s, src, dst, i in steps:
        buf = s % 2
        if s == 0:
            in_copies[s] = pltpu.async_copy(
                src.at[:, pl.ds(chunk_base(i), TCH)], colbufs[buf], sem_in)
        in_copies[s].wait()
        if s + 1 < len(steps):
            ns, nsrc, _, ni = steps[s + 1]
            in_copies[ns] = pltpu.async_copy(
                nsrc.at[:, pl.ds(chunk_base(ni), TCH)], colbufs[ns % 2], sem_in)
        if s >= 2:
            out_copies[s - 2].wait()

        def tile16(g, carry, buf=buf):
            r0 = g * EMB_K
            for dr in range(EMB_K):
                r = r0 + dr
                row = plsc.load_gather(
                    colbufs[buf], [jlanes, jnp.full((EMB_K,), 0, jnp.int32) + r])
                rowbufs[buf][r] = row
            return carry

        lax.fori_loop(0, TCH // EMB_K, tile16, 0)
        out_copies[s] = pltpu.async_copy(
            rowbufs[buf], dst.at[pl.ds(chunk_base(i), TCH)], out_sems[buf])
    out_copies[len(steps) - 2].wait()
    out_copies[len(steps) - 1].wait()


def _sc_gather_body(a_hbm, b_hbm, uidx_hbm, vidx_hbm, u_hbm, v_hbm,
                    uidx_v, vidx_v, arows_v, brows_v, sem_a, sem_b):
    wid = lax.axis_index("s") * NC + lax.axis_index("c")
    base = wid * BPW

    pltpu.sync_copy(uidx_hbm.at[wid], uidx_v)
    pltpu.sync_copy(vidx_hbm.at[wid], vidx_v)

    copies = []
    for j in range(NIDX):
        dst = pl.ds(j * IDX_BLK, IDX_BLK)
        copies.append(pltpu.async_copy(a_hbm.at[uidx_v.at[j]], arows_v.at[dst], sem_a))
        copies.append(pltpu.async_copy(b_hbm.at[vidx_v.at[j]], brows_v.at[dst], sem_b))
    for c in copies:
        c.wait()

    pltpu.sync_copy(arows_v, u_hbm.at[pl.ds(base, BPW)])
    pltpu.sync_copy(brows_v, v_hbm.at[pl.ds(base, BPW)])


@functools.cache
def _get_sc_transpose():
    dst_sds = jax.ShapeDtypeStruct((NUM_ROWS, EMB_K), jnp.float32)
    return pl.kernel(
        _sc_transpose_body,
        out_type=(dst_sds, dst_sds),
        mesh=plsc.VectorSubcoreMesh(core_axis_name="c", subcore_axis_name="s",
                                    num_cores=NC, num_subcores=NS),
        compiler_params=pltpu.CompilerParams(needs_layout_passes=False,
                                             use_tc_tiling_on_sc=False),
        scratch_types=[
            pltpu.VMEM((EMB_K, TCH), jnp.float32),
            pltpu.VMEM((EMB_K, TCH), jnp.float32),
            pltpu.VMEM((TCH, EMB_K), jnp.float32),
            pltpu.VMEM((TCH, EMB_K), jnp.float32),
            pltpu.SemaphoreType.DMA,
            pltpu.SemaphoreType.DMA,
            pltpu.SemaphoreType.DMA,
        ],
    )


@functools.cache
def _get_sc_gather():
    emb_sds = jax.ShapeDtypeStruct((BATCH, EMB_K), jnp.float32)
    return pl.kernel(
        _sc_gather_body,
        out_type=(emb_sds, emb_sds),
        mesh=plsc.VectorSubcoreMesh(core_axis_name="c", subcore_axis_name="s",
                                    num_cores=NC, num_subcores=NS),
        compiler_params=pltpu.CompilerParams(needs_layout_passes=False,
                                             use_tc_tiling_on_sc=False),
        scratch_types=[
            pltpu.VMEM((NIDX, IDX_BLK), jnp.int32),
            pltpu.VMEM((NIDX, IDX_BLK), jnp.int32),
            pltpu.VMEM((BPW, EMB_K), jnp.float32),
            pltpu.VMEM((BPW, EMB_K), jnp.float32),
            pltpu.SemaphoreType.DMA,
            pltpu.SemaphoreType.DMA,
        ],
    )


def _mlp_body(u_ref, v_ref, bdu_ref, bdv_ref, b1t_ref, bd2_ref, o_ref):
    h = (
        jnp.dot(u_ref[...], bdu_ref[...], preferred_element_type=jnp.float32)
        + jnp.dot(v_ref[...], bdv_ref[...], preferred_element_type=jnp.float32)
        + b1t_ref[...]
    )
    h = jnp.maximum(h, 0.0)
    o_ref[...] = jnp.dot(h, bd2_ref[...], preferred_element_type=jnp.float32)


def _mlp(u2, v2, bdu, bdv, b1t, bd2):
    blk = pl.BlockSpec((MLP_BLK, 128), lambda i: (i, 0))
    w128 = pl.BlockSpec((128, 128), lambda i: (0, 0))
    row = pl.BlockSpec((1, 128), lambda i: (0, 0))
    w8 = pl.BlockSpec((128, PACK), lambda i: (0, 0))
    oblk = pl.BlockSpec((MLP_BLK, PACK), lambda i: (i, 0))
    return pl.pallas_call(
        _mlp_body,
        grid=(PB // MLP_BLK,),
        in_specs=[blk, blk, w128, w128, row, w8],
        out_specs=oblk,
        out_shape=jax.ShapeDtypeStruct((PB, PACK), jnp.float32),
    )(u2, v2, bdu, bdv, b1t, bd2)


@jax.jit
def _ncf_forward(x, W, H, W1, b1, W2):
    A, B = _get_sc_transpose()(W.T, H.T)

    uidx = x[:, 0].astype(jnp.int32).reshape(NW, NIDX, IDX_BLK)
    vidx = x[:, 1].astype(jnp.int32).reshape(NW, NIDX, IDX_BLK)
    U, V = _get_sc_gather()(A, B, uidx, vidx)

    w1ut = W1[:, :EMB_K].T
    w1vt = W1[:, EMB_K:].T
    eye = jnp.eye(PACK, dtype=jnp.float32)
    bdu = jnp.einsum("tb,jk->tjbk", eye, w1ut).reshape(128, 128)
    bdv = jnp.einsum("tb,jk->tjbk", eye, w1vt).reshape(128, 128)
    b1t = jnp.tile(b1.reshape(1, EMB_K), (1, PACK))
    bd2 = jnp.einsum("tb,k->tkb", eye, W2.reshape(EMB_K)).reshape(128, PACK)

    out = _mlp(U.reshape(PB, 128), V.reshape(PB, 128), bdu, bdv, b1t, bd2)
    return out.reshape(BATCH, 1)


def kernel(x, W, H, W1, b1, W2):
    return _ncf_forward(x, W, H, W1, b1, W2)


# speedup vs baseline: 2.3187x; 1.4979x over previous
import functools

import jax
import jax.numpy as jnp
from jax import lax
from jax.experimental import pallas as pl
from jax.experimental.pallas import tpu as pltpu
from jax.experimental.pallas import tpu_sc as plsc

NUM_ROWS = 100000
EMB_K = 16
BATCH = 16384

NC = 2
NS = 16
NW = NC * NS
BPW = BATCH // NW
IDX_BLK = 128
NIDX = BPW // IDX_BLK

TCH = 800
NCHUNK = NUM_ROWS // TCH
CPW = -(-NCHUNK // NW)

PACK = 128 // EMB_K
PB = BATCH // PACK
MLP_BLK = 512


def _sc_transpose_body(wt_hbm, ht_hbm, a_hbm, b_hbm,
                       colbuf0, colbuf1, rowbuf0, rowbuf1,
                       sem_in, sem_out0, sem_out1):
    wid = lax.axis_index("s") * NC + lax.axis_index("c")
    jlanes = lax.iota(jnp.int32, EMB_K)
    colbufs = (colbuf0, colbuf1)
    rowbufs = (rowbuf0, rowbuf1)
    out_sems = (sem_out0, sem_out1)

    steps = []
    for t, (src, dst) in enumerate(((wt_hbm, a_hbm), (ht_hbm, b_hbm))):
        for i in range(CPW):
            steps.append((t * CPW + i, src, dst, i))

    def chunk_base(i):
        c = i * NW + wid
        c = jnp.where(c < NCHUNK, c, 0)
        return c * TCH

    in_copies = {}
    out_copies = {}
    for s, src, dst, i in steps:
        buf = s % 2
        if s == 0:
            in_copies[s] = pltpu.async_copy(
                src.at[:, pl.ds(chunk_base(i), TCH)],
                colbufs[buf].at[:, pl.ds(0, TCH)], sem_in)
        in_copies[s].wait()
        if s + 1 < len(steps):
            ns, nsrc, _, ni = steps[s + 1]
            in_copies[ns] = pltpu.async_copy(
                nsrc.at[:, pl.ds(chunk_base(ni), TCH)],
                colbufs[ns % 2].at[:, pl.ds(0, TCH)], sem_in)
        if s >= 2:
            out_copies[s - 2].wait()

        def tile16(g, carry, buf=buf):
            r0 = g * EMB_K
            for dr in range(EMB_K):
                r = r0 + dr
                row = plsc.load_gather(
                    colbufs[buf], [jlanes, jnp.full((EMB_K,), 0, jnp.int32) + r])
                rowbufs[buf][r] = row
            return carry

        lax.fori_loop(0, TCH // EMB_K, tile16, 0)
        out_copies[s] = pltpu.async_copy(
            rowbufs[buf], dst.at[pl.ds(chunk_base(i), TCH)], out_sems[buf])
    out_copies[len(steps) - 2].wait()
    out_copies[len(steps) - 1].wait()


def _sc_gather_body(a_hbm, b_hbm, uidx_hbm, vidx_hbm, u_hbm, v_hbm,
                    uidx_v, vidx_v, arows_v, brows_v, sem_a, sem_b):
    wid = lax.axis_index("s") * NC + lax.axis_index("c")
    base = wid * BPW

    pltpu.sync_copy(uidx_hbm.at[wid], uidx_v)
    pltpu.sync_copy(vidx_hbm.at[wid], vidx_v)

    copies = []
    for j in range(NIDX):
        dst = pl.ds(j * IDX_BLK, IDX_BLK)
        copies.append(pltpu.async_copy(a_hbm.at[uidx_v.at[j]], arows_v.at[dst], sem_a))
        copies.append(pltpu.async_copy(b_hbm.at[vidx_v.at[j]], brows_v.at[dst], sem_b))
    for c in copies:
        c.wait()

    pltpu.sync_copy(arows_v, u_hbm.at[pl.ds(base, BPW)])
    pltpu.sync_copy(brows_v, v_hbm.at[pl.ds(base, BPW)])


@functools.cache
def _get_sc_transpose():
    dst_sds = jax.ShapeDtypeStruct((NUM_ROWS, EMB_K), jnp.float32)
    return pl.kernel(
        _sc_transpose_body,
        out_type=(dst_sds, dst_sds),
        mesh=plsc.VectorSubcoreMesh(core_axis_name="c", subcore_axis_name="s",
                                    num_cores=NC, num_subcores=NS),
        compiler_params=pltpu.CompilerParams(needs_layout_passes=False,
                                             use_tc_tiling_on_sc=False),
        scratch_types=[
            pltpu.VMEM((EMB_K, TCH + 1), jnp.float32),
            pltpu.VMEM((EMB_K, TCH + 1), jnp.float32),
            pltpu.VMEM((TCH, EMB_K), jnp.float32),
            pltpu.VMEM((TCH, EMB_K), jnp.float32),
            pltpu.SemaphoreType.DMA,
            pltpu.SemaphoreType.DMA,
            pltpu.SemaphoreType.DMA,
        ],
    )


@functools.cache
def _get_sc_gather():
    emb_sds = jax.ShapeDtypeStruct((BATCH, EMB_K), jnp.float32)
    return pl.kernel(
        _sc_gather_body,
        out_type=(emb_sds, emb_sds),
        mesh=plsc.VectorSubcoreMesh(core_axis_name="c", subcore_axis_name="s",
                                    num_cores=NC, num_subcores=NS),
        compiler_params=pltpu.CompilerParams(needs_layout_passes=False,
                                             use_tc_tiling_on_sc=False),
        scratch_types=[
            pltpu.VMEM((NIDX, IDX_BLK), jnp.int32),
            pltpu.VMEM((NIDX, IDX_BLK), jnp.int32),
            pltpu.VMEM((BPW, EMB_K), jnp.float32),
            pltpu.VMEM((BPW, EMB_K), jnp.float32),
            pltpu.SemaphoreType.DMA,
            pltpu.SemaphoreType.DMA,
        ],
    )


def _mlp_body(u_ref, v_ref, bdu_ref, bdv_ref, b1t_ref, bd2_ref, o_ref):
    h = (
        jnp.dot(u_ref[...], bdu_ref[...], preferred_element_type=jnp.float32)
        + jnp.dot(v_ref[...], bdv_ref[...], preferred_element_type=jnp.float32)
        + b1t_ref[...]
    )
    h = jnp.maximum(h, 0.0)
    o_ref[...] = jnp.dot(h, bd2_ref[...], preferred_element_type=jnp.float32)


def _mlp(u2, v2, bdu, bdv, b1t, bd2):
    blk = pl.BlockSpec((MLP_BLK, 128), lambda i: (i, 0))
    w128 = pl.BlockSpec((128, 128), lambda i: (0, 0))
    row = pl.BlockSpec((1, 128), lambda i: (0, 0))
    w8 = pl.BlockSpec((128, PACK), lambda i: (0, 0))
    oblk = pl.BlockSpec((MLP_BLK, PACK), lambda i: (i, 0))
    return pl.pallas_call(
        _mlp_body,
        grid=(PB // MLP_BLK,),
        in_specs=[blk, blk, w128, w128, row, w8],
        out_specs=oblk,
        out_shape=jax.ShapeDtypeStruct((PB, PACK), jnp.float32),
    )(u2, v2, bdu, bdv, b1t, bd2)


@jax.jit
def _ncf_forward(x, W, H, W1, b1, W2):
    A, B = _get_sc_transpose()(W.T, H.T)

    uidx = x[:, 0].astype(jnp.int32).reshape(NW, NIDX, IDX_BLK)
    vidx = x[:, 1].astype(jnp.int32).reshape(NW, NIDX, IDX_BLK)
    U, V = _get_sc_gather()(A, B, uidx, vidx)

    w1ut = W1[:, :EMB_K].T
    w1vt = W1[:, EMB_K:].T
    eye = jnp.eye(PACK, dtype=jnp.float32)
    bdu = jnp.einsum("tb,jk->tjbk", eye, w1ut).reshape(128, 128)
    bdv = jnp.einsum("tb,jk->tjbk", eye, w1vt).reshape(128, 128)
    b1t = jnp.tile(b1.reshape(1, EMB_K), (1, PACK))
    bd2 = jnp.einsum("tb,k->tkb", eye, W2.reshape(EMB_K)).reshape(128, PACK)

    out = _mlp(U.reshape(PB, 128), V.reshape(PB, 128), bdu, bdv, b1t, bd2)
    return out.reshape(BATCH, 1)


def kernel(x, W, H, W1, b1, W2):
    return _ncf_forward(x, W, H, W1, b1, W2)


# speedup vs baseline: 2.8999x; 1.2507x over previous
import functools

import jax
import jax.numpy as jnp
from jax import lax
from jax.experimental import pallas as pl
from jax.experimental.pallas import tpu as pltpu
from jax.experimental.pallas import tpu_sc as plsc

NUM_ROWS = 100000
EMB_K = 16
BATCH = 16384

NC = 2
NS = 16
NW = NC * NS
BPW = BATCH // NW
IDX_BLK = 128
NIDX = BPW // IDX_BLK

TCH = 800
NCHUNK = NUM_ROWS // TCH
CPW = -(-NCHUNK // NW)

PACK = 128 // EMB_K
PB = BATCH // PACK
MLP_BLK = 512


def _sc_transpose_body(wt_hbm, ht_hbm, a_hbm, b_hbm,
                       colbuf0, colbuf1, rowbuf0, rowbuf1,
                       sem_in, sem_out0, sem_out1):
    wid = lax.axis_index("s") * NC + lax.axis_index("c")
    jlanes = lax.iota(jnp.int32, EMB_K)
    colbufs = (colbuf0, colbuf1)
    rowbufs = (rowbuf0, rowbuf1)
    out_sems = (sem_out0, sem_out1)

    steps = []
    for t, (src, dst) in enumerate(((wt_hbm, a_hbm), (ht_hbm, b_hbm))):
        for i in range(CPW):
            steps.append((t * CPW + i, src, dst, i))

    def chunk_base(i):
        c = i * NW + wid
        c = jnp.where(c < NCHUNK, c, 0)
        return c * TCH

    in_copies = {}
    out_copies = {}
    for s, src, dst, i in steps:
        buf = s % 2
        if s == 0:
            in_copies[s] = pltpu.async_copy(
                src.at[:, pl.ds(chunk_base(i), TCH)],
                colbufs[buf].at[:, pl.ds(0, TCH)], sem_in)
        in_copies[s].wait()
        if s + 1 < len(steps):
            ns, nsrc, _, ni = steps[s + 1]
            in_copies[ns] = pltpu.async_copy(
                nsrc.at[:, pl.ds(chunk_base(ni), TCH)],
                colbufs[ns % 2].at[:, pl.ds(0, TCH)], sem_in)
        if s >= 2:
            out_copies[s - 2].wait()

        @plsc.parallel_loop(0, TCH // EMB_K, unroll=4)
        def tile16(g, buf=buf):
            r0 = g * EMB_K
            for dr in range(EMB_K):
                r = r0 + dr
                row = plsc.load_gather(
                    colbufs[buf], [jlanes, jnp.full((EMB_K,), 0, jnp.int32) + r])
                rowbufs[buf][r] = row
        out_copies[s] = pltpu.async_copy(
            rowbufs[buf], dst.at[pl.ds(chunk_base(i), TCH)], out_sems[buf])
    out_copies[len(steps) - 2].wait()
    out_copies[len(steps) - 1].wait()


def _sc_gather_body(a_hbm, b_hbm, uidx_hbm, vidx_hbm, u_hbm, v_hbm,
                    uidx_v, vidx_v, arows_v, brows_v, sem_a, sem_b):
    wid = lax.axis_index("s") * NC + lax.axis_index("c")
    base = wid * BPW

    pltpu.sync_copy(uidx_hbm.at[wid], uidx_v)
    pltpu.sync_copy(vidx_hbm.at[wid], vidx_v)

    copies = []
    for j in range(NIDX):
        dst = pl.ds(j * IDX_BLK, IDX_BLK)
        copies.append(pltpu.async_copy(a_hbm.at[uidx_v.at[j]], arows_v.at[dst], sem_a))
        copies.append(pltpu.async_copy(b_hbm.at[vidx_v.at[j]], brows_v.at[dst], sem_b))
    for c in copies:
        c.wait()

    pltpu.sync_copy(arows_v, u_hbm.at[pl.ds(base, BPW)])
    pltpu.sync_copy(brows_v, v_hbm.at[pl.ds(base, BPW)])


@functools.cache
def _get_sc_transpose():
    dst_sds = jax.ShapeDtypeStruct((NUM_ROWS, EMB_K), jnp.float32)
    return pl.kernel(
        _sc_transpose_body,
        out_type=(dst_sds, dst_sds),
        mesh=plsc.VectorSubcoreMesh(core_axis_name="c", subcore_axis_name="s",
                                    num_cores=NC, num_subcores=NS),
        compiler_params=pltpu.CompilerParams(needs_layout_passes=False,
                                             use_tc_tiling_on_sc=False),
        scratch_types=[
            pltpu.VMEM((EMB_K, TCH + 1), jnp.float32),
            pltpu.VMEM((EMB_K, TCH + 1), jnp.float32),
            pltpu.VMEM((TCH, EMB_K), jnp.float32),
            pltpu.VMEM((TCH, EMB_K), jnp.float32),
            pltpu.SemaphoreType.DMA,
            pltpu.SemaphoreType.DMA,
            pltpu.SemaphoreType.DMA,
        ],
    )


@functools.cache
def _get_sc_gather():
    emb_sds = jax.ShapeDtypeStruct((BATCH, EMB_K), jnp.float32)
    return pl.kernel(
        _sc_gather_body,
        out_type=(emb_sds, emb_sds),
        mesh=plsc.VectorSubcoreMesh(core_axis_name="c", subcore_axis_name="s",
                                    num_cores=NC, num_subcores=NS),
        compiler_params=pltpu.CompilerParams(needs_layout_passes=False,
                                             use_tc_tiling_on_sc=False),
        scratch_types=[
            pltpu.VMEM((NIDX, IDX_BLK), jnp.int32),
            pltpu.VMEM((NIDX, IDX_BLK), jnp.int32),
            pltpu.VMEM((BPW, EMB_K), jnp.float32),
            pltpu.VMEM((BPW, EMB_K), jnp.float32),
            pltpu.SemaphoreType.DMA,
            pltpu.SemaphoreType.DMA,
        ],
    )


def _mlp_body(u_ref, v_ref, bdu_ref, bdv_ref, b1t_ref, bd2_ref, o_ref):
    h = (
        jnp.dot(u_ref[...], bdu_ref[...], preferred_element_type=jnp.float32)
        + jnp.dot(v_ref[...], bdv_ref[...], preferred_element_type=jnp.float32)
        + b1t_ref[...]
    )
    h = jnp.maximum(h, 0.0)
    o_ref[...] = jnp.dot(h, bd2_ref[...], preferred_element_type=jnp.float32)


def _mlp(u2, v2, bdu, bdv, b1t, bd2):
    blk = pl.BlockSpec((MLP_BLK, 128), lambda i: (i, 0))
    w128 = pl.BlockSpec((128, 128), lambda i: (0, 0))
    row = pl.BlockSpec((1, 128), lambda i: (0, 0))
    w8 = pl.BlockSpec((128, PACK), lambda i: (0, 0))
    oblk = pl.BlockSpec((MLP_BLK, PACK), lambda i: (i, 0))
    return pl.pallas_call(
        _mlp_body,
        grid=(PB // MLP_BLK,),
        in_specs=[blk, blk, w128, w128, row, w8],
        out_specs=oblk,
        out_shape=jax.ShapeDtypeStruct((PB, PACK), jnp.float32),
    )(u2, v2, bdu, bdv, b1t, bd2)


@jax.jit
def _ncf_forward(x, W, H, W1, b1, W2):
    A, B = _get_sc_transpose()(W.T, H.T)

    uidx = x[:, 0].astype(jnp.int32).reshape(NW, NIDX, IDX_BLK)
    vidx = x[:, 1].astype(jnp.int32).reshape(NW, NIDX, IDX_BLK)
    U, V = _get_sc_gather()(A, B, uidx, vidx)

    w1ut = W1[:, :EMB_K].T
    w1vt = W1[:, EMB_K:].T
    eye = jnp.eye(PACK, dtype=jnp.float32)
    bdu = jnp.einsum("tb,jk->tjbk", eye, w1ut).reshape(128, 128)
    bdv = jnp.einsum("tb,jk->tjbk", eye, w1vt).reshape(128, 128)
    b1t = jnp.tile(b1.reshape(1, EMB_K), (1, PACK))
    bd2 = jnp.einsum("tb,k->tkb", eye, W2.reshape(EMB_K)).reshape(128, PACK)

    out = _mlp(U.reshape(PB, 128), V.reshape(PB, 128), bdu, bdv, b1t, bd2)
    return out.reshape(BATCH, 1)


def kernel(x, W, H, W1, b1, W2):
    return _ncf_forward(x, W, H, W1, b1, W2)
